# Initial kernel scaffold; baseline (speedup 1.0000x reference)
import jax
import jax.numpy as jnp
from jax.experimental import pallas as pl


def kernel(input_vector, embedding, W_ih_0, W_hh_0, b_ih_0, b_hh_0, W_ih_1, W_hh_1, b_ih_1, b_hh_1, W_ih_2, W_hh_2, b_ih_2, b_hh_2, W_lin, b_lin):
    raise NotImplementedError("write your pallas kernel here")


# speedup vs baseline: 3.9074x; 3.9074x over previous
import functools

import jax
import jax.numpy as jnp
from jax import lax
from jax.experimental import pallas as pl
from jax.experimental.pallas import tpu as pltpu
from jax.experimental.pallas import tpu_sc as plsc

VOC = 1000
H = 512
E = 256
B = 128
T = 50
G4 = 4 * H
VP = 1024
CH = 5
NW = 32
ROWS = B * T
RPW = ROWS // NW


def _lstm_body(x0_ref, wi0_ref, wh0_ref, b0_ref,
               wi1_ref, wh1_ref, b1_ref,
               wi2_ref, wh2_ref, b2_ref,
               ys_ref, hn_ref, cn_ref, g_ref):
    def run_layer(src_ref, wi_ref, wh_ref, b_ref, dst_ref, lidx):
        wi = wi_ref[...]
        wh = wh_ref[...]
        b = b_ref[...]

        def chunk(ci, carry):
            h0, c0 = carry
            xc = src_ref[pl.ds(ci * CH, CH)]
            g_ref[...] = jnp.dot(xc.reshape(CH * B, xc.shape[-1]), wi,
                                 preferred_element_type=jnp.float32) + b

            def step(i, carry):
                h, c = carry
                gates = g_ref[pl.ds(i * B, B), :]
                gates = gates + jnp.dot(h, wh, preferred_element_type=jnp.float32)
                ii = jax.nn.sigmoid(gates[:, 0 * H:1 * H])
                ff = jax.nn.sigmoid(gates[:, 1 * H:2 * H])
                gg = jnp.tanh(gates[:, 2 * H:3 * H])
                oo = jax.nn.sigmoid(gates[:, 3 * H:4 * H])
                c = ff * c + ii * gg
                h = oo * jnp.tanh(c)
                dst_ref[ci * CH + i] = h
                return (h, c)

            return lax.fori_loop(0, CH, step, (h0, c0))

        z = jnp.zeros((B, H), jnp.float32)
        h, c = lax.fori_loop(0, T // CH, chunk, (z, z))
        hn_ref[lidx] = h
        cn_ref[lidx] = c

    run_layer(x0_ref, wi0_ref, wh0_ref, b0_ref, ys_ref, 0)
    run_layer(ys_ref, wi1_ref, wh1_ref, b1_ref, ys_ref, 1)
    run_layer(ys_ref, wi2_ref, wh2_ref, b2_ref, ys_ref, 2)


def _lstm_call(x0, wi0, wh0, b0, wi1, wh1, b1, wi2, wh2, b2):
    return pl.pallas_call(
        _lstm_body,
        out_shape=[
            jax.ShapeDtypeStruct((T, B, H), jnp.float32),
            jax.ShapeDtypeStruct((3, B, H), jnp.float32),
            jax.ShapeDtypeStruct((3, B, H), jnp.float32),
        ],
        scratch_shapes=[pltpu.VMEM((CH * B, G4), jnp.float32)],
    )(x0, wi0, wh0, b0, wi1, wh1, b1, wi2, wh2, b2)


def _proj_body(x_ref, w_ref, b_ref, o_ref):
    o_ref[...] = jnp.dot(x_ref[...], w_ref[...],
                         preferred_element_type=jnp.float32) + b_ref[...]


def _proj_call(ys_flat, wlin, blin):
    rows_per = 800
    grid = ROWS // rows_per
    return pl.pallas_call(
        _proj_body,
        grid=(grid,),
        in_specs=[
            pl.BlockSpec((rows_per, H), lambda i: (i, 0)),
            pl.BlockSpec((H, VP), lambda i: (0, 0)),
            pl.BlockSpec((1, VP), lambda i: (0, 0)),
        ],
        out_specs=pl.BlockSpec((rows_per, VP), lambda i: (i, 0)),
        out_shape=jax.ShapeDtypeStruct((ROWS, VP), jnp.float32),
    )(ys_flat, wlin, blin)


def _gather_body(table_hbm, idx_hbm, out_hbm, idx_v, rows_v, sem):
    wid = lax.axis_index("s") * 2 + lax.axis_index("c")
    base = wid * RPW
    pltpu.sync_copy(idx_hbm.at[pl.ds(base, RPW)], idx_v)
    pltpu.async_copy(table_hbm.at[idx_v], rows_v, sem).wait()
    pltpu.sync_copy(rows_v, out_hbm.at[pl.ds(base, RPW)])


def _sc_gather(table, idx):
    mesh = plsc.VectorSubcoreMesh(core_axis_name="c", subcore_axis_name="s")
    k = functools.partial(
        pl.kernel, mesh=mesh,
        out_type=jax.ShapeDtypeStruct((ROWS, E), jnp.float32),
        scratch_types=[
            pltpu.VMEM((RPW,), jnp.int32),
            pltpu.VMEM((RPW, E), jnp.float32),
            pltpu.SemaphoreType.DMA,
        ],
    )(_gather_body)
    return k(table, idx)


def kernel(input_vector, embedding,
           W_ih_0, W_hh_0, b_ih_0, b_hh_0,
           W_ih_1, W_hh_1, b_ih_1, b_hh_1,
           W_ih_2, W_hh_2, b_ih_2, b_hh_2,
           W_lin, b_lin):
    idx_tm = input_vector.T.reshape(-1)
    x0 = _sc_gather(embedding, idx_tm).reshape(T, B, E)

    wi0, wh0 = W_ih_0.T, W_hh_0.T
    wi1, wh1 = W_ih_1.T, W_hh_1.T
    wi2, wh2 = W_ih_2.T, W_hh_2.T
    b0 = (b_ih_0 + b_hh_0).reshape(1, G4)
    b1 = (b_ih_1 + b_hh_1).reshape(1, G4)
    b2 = (b_ih_2 + b_hh_2).reshape(1, G4)

    ys, h_n, c_n = _lstm_call(x0, wi0, wh0, b0, wi1, wh1, b1, wi2, wh2, b2)

    wlin = jnp.pad(W_lin.T, ((0, 0), (0, VP - VOC)))
    blin = jnp.pad(b_lin, (0, VP - VOC)).reshape(1, VP)
    out_tm = _proj_call(ys.reshape(ROWS, H), wlin, blin)

    output_data = out_tm.reshape(T, B, VP)[:, :, :VOC].transpose(1, 0, 2)
    return output_data, h_n, c_n


# speedup vs baseline: 4.1796x; 1.0697x over previous
import functools

import jax
import jax.numpy as jnp
from jax import lax
from jax.experimental import pallas as pl
from jax.experimental.pallas import tpu as pltpu
from jax.experimental.pallas import tpu_sc as plsc

VOC = 1000
H = 512
E = 256
B = 128
T = 50
G4 = 4 * H
VP = 1024
CH = 5
NW = 32
ROWS = B * T
RPW = ROWS // NW


def _lstm_body(x0_ref, wi0_ref, wh0_ref, b0_ref,
               wi1_ref, wh1_ref, b1_ref,
               wi2_ref, wh2_ref, b2_ref,
               ys_ref, hn_ref, cn_ref, g_ref):
    def run_layer(src_ref, wi_ref, wh_ref, b_ref, dst_ref, lidx):
        wi = wi_ref[...]
        wh = wh_ref[...]
        b = b_ref[...]

        def chunk(ci, carry):
            h0, c0 = carry
            xc = src_ref[pl.ds(ci * CH, CH)]
            xb = xc.reshape(CH * B, xc.shape[-1]).astype(jnp.bfloat16)
            g_ref[...] = jnp.dot(xb, wi, preferred_element_type=jnp.float32) + b

            def step(i, carry):
                h, c = carry
                gates = g_ref[pl.ds(i * B, B), :]
                gates = gates + jnp.dot(h.astype(jnp.bfloat16), wh,
                                        preferred_element_type=jnp.float32)
                ii = jax.nn.sigmoid(gates[:, 0 * H:1 * H])
                ff = jax.nn.sigmoid(gates[:, 1 * H:2 * H])
                gg = jnp.tanh(gates[:, 2 * H:3 * H])
                oo = jax.nn.sigmoid(gates[:, 3 * H:4 * H])
                c = ff * c + ii * gg
                h = oo * jnp.tanh(c)
                dst_ref[ci * CH + i] = h
                return (h, c)

            return lax.fori_loop(0, CH, step, (h0, c0))

        z = jnp.zeros((B, H), jnp.float32)
        h, c = lax.fori_loop(0, T // CH, chunk, (z, z))
        hn_ref[lidx] = h
        cn_ref[lidx] = c

    run_layer(x0_ref, wi0_ref, wh0_ref, b0_ref, ys_ref, 0)
    run_layer(ys_ref, wi1_ref, wh1_ref, b1_ref, ys_ref, 1)
    run_layer(ys_ref, wi2_ref, wh2_ref, b2_ref, ys_ref, 2)


def _lstm_call(x0, wi0, wh0, b0, wi1, wh1, b1, wi2, wh2, b2):
    return pl.pallas_call(
        _lstm_body,
        out_shape=[
            jax.ShapeDtypeStruct((T, B, H), jnp.float32),
            jax.ShapeDtypeStruct((3, B, H), jnp.float32),
            jax.ShapeDtypeStruct((3, B, H), jnp.float32),
        ],
        scratch_shapes=[pltpu.VMEM((CH * B, G4), jnp.float32)],
    )(x0, wi0, wh0, b0, wi1, wh1, b1, wi2, wh2, b2)


def _proj_body(x_ref, w_ref, b_ref, o_ref):
    o_ref[...] = jnp.dot(x_ref[...].astype(jnp.bfloat16), w_ref[...],
                         preferred_element_type=jnp.float32) + b_ref[...]


def _proj_call(ys_flat, wlin, blin):
    rows_per = 800
    grid = ROWS // rows_per
    return pl.pallas_call(
        _proj_body,
        grid=(grid,),
        in_specs=[
            pl.BlockSpec((rows_per, H), lambda i: (i, 0)),
            pl.BlockSpec((H, VP), lambda i: (0, 0)),
            pl.BlockSpec((1, VP), lambda i: (0, 0)),
        ],
        out_specs=pl.BlockSpec((rows_per, VP), lambda i: (i, 0)),
        out_shape=jax.ShapeDtypeStruct((ROWS, VP), jnp.float32),
    )(ys_flat, wlin, blin)


def _gather_body(table_hbm, idx_hbm, out_hbm, idx_v, rows_v, sem):
    wid = lax.axis_index("s") * 2 + lax.axis_index("c")
    base = wid * RPW
    pltpu.sync_copy(idx_hbm.at[pl.ds(base, RPW)], idx_v)
    pltpu.async_copy(table_hbm.at[idx_v], rows_v, sem).wait()
    pltpu.sync_copy(rows_v, out_hbm.at[pl.ds(base, RPW)])


def _sc_gather(table, idx):
    mesh = plsc.VectorSubcoreMesh(core_axis_name="c", subcore_axis_name="s")
    k = functools.partial(
        pl.kernel, mesh=mesh,
        out_type=jax.ShapeDtypeStruct((ROWS, E), jnp.float32),
        scratch_types=[
            pltpu.VMEM((RPW,), jnp.int32),
            pltpu.VMEM((RPW, E), jnp.float32),
            pltpu.SemaphoreType.DMA,
        ],
    )(_gather_body)
    return k(table, idx)


def kernel(input_vector, embedding,
           W_ih_0, W_hh_0, b_ih_0, b_hh_0,
           W_ih_1, W_hh_1, b_ih_1, b_hh_1,
           W_ih_2, W_hh_2, b_ih_2, b_hh_2,
           W_lin, b_lin):
    idx_tm = input_vector.T.reshape(-1)
    x0 = _sc_gather(embedding, idx_tm).reshape(T, B, E)

    bf = jnp.bfloat16
    wi0, wh0 = W_ih_0.T.astype(bf), W_hh_0.T.astype(bf)
    wi1, wh1 = W_ih_1.T.astype(bf), W_hh_1.T.astype(bf)
    wi2, wh2 = W_ih_2.T.astype(bf), W_hh_2.T.astype(bf)
    b0 = (b_ih_0 + b_hh_0).reshape(1, G4)
    b1 = (b_ih_1 + b_hh_1).reshape(1, G4)
    b2 = (b_ih_2 + b_hh_2).reshape(1, G4)

    ys, h_n, c_n = _lstm_call(x0, wi0, wh0, b0, wi1, wh1, b1, wi2, wh2, b2)

    wlin = jnp.pad(W_lin.T, ((0, 0), (0, VP - VOC))).astype(bf)
    blin = jnp.pad(b_lin, (0, VP - VOC)).reshape(1, VP)
    out_tm = _proj_call(ys.reshape(ROWS, H), wlin, blin)

    output_data = out_tm.reshape(T, B, VP)[:, :, :VOC].transpose(1, 0, 2)
    return output_data, h_n, c_n
